# baseline (speedup 1.0000x reference)
import functools

import jax
import jax.numpy as jnp
from jax import lax
from jax.experimental import pallas as pl
from jax.experimental.pallas import tpu as pltpu
from jax.experimental.pallas import tpu_sc as plsc

_NUM_ROWS = 4096
_EMB = 64
_NUM_COLS = 3 * _EMB
_LINE = 2 * _EMB
_NC = 2
_NS = 16
_NW = _NC * _NS
_B = _NUM_ROWS // _NW
_NG = _B // 16


@functools.partial(
    pl.kernel,
    mesh=plsc.VectorSubcoreMesh(core_axis_name="c", subcore_axis_name="s"),
    out_type=jax.ShapeDtypeStruct((_NUM_ROWS, _NUM_COLS), jnp.float32),
    scratch_types=[
        pltpu.VMEM((_B,), jnp.int32),
        pltpu.VMEM((_B,), jnp.int32),
        pltpu.VMEM((_B,), jnp.int32),
        pltpu.VMEM((_B,), jnp.int32),
        pltpu.VMEM((_B, _LINE), jnp.float32),
        pltpu.VMEM((_B, _LINE), jnp.float32),
        pltpu.VMEM((_B, _LINE), jnp.float32),
        pltpu.VMEM((_B, _LINE), jnp.float32),
        pltpu.VMEM((_B, _NUM_COLS), jnp.float32),
        pltpu.SemaphoreType.DMA,
    ],
)
def _emb_kernel(heads_hbm, rels_hbm, tails_hbm, names_hbm, specials_hbm,
                hid_hbm, rid_hbm, tid_hbm, nid_hbm, out_hbm,
                hid_v, rid_v, tid_v, nid_v,
                h_v, r_v, t_v, n_v, out_v, sem):
    wid = lax.axis_index("s") * _NC + lax.axis_index("c")
    base = wid * _B

    pltpu.sync_copy(hid_hbm.at[pl.ds(base, _B)], hid_v)
    pltpu.sync_copy(rid_hbm.at[pl.ds(base, _B)], rid_v)
    pltpu.sync_copy(tid_hbm.at[pl.ds(base, _B)], tid_v)
    pltpu.sync_copy(nid_hbm.at[pl.ds(base, _B)], nid_v)

    def issue_body(g, carry):
        e0 = g * 16
        hv = hid_v[pl.ds(e0, 16)]
        rv = rid_v[pl.ds(e0, 16)]
        tv = tid_v[pl.ds(e0, 16)]
        nv = nid_v[pl.ds(e0, 16)]
        hmv, hsv = hv >> 4, (hv >> 1) & 7
        rmv, rsv = rv >> 4, (rv >> 1) & 7
        tmv, tsv = tv >> 4, (tv >> 1) & 7
        nmv, nsv = nv >> 4, (nv >> 1) & 7
        for j in range(16):
            e = e0 + j
            pltpu.async_copy(heads_hbm.at[hmv[j], hsv[j]], h_v.at[e], sem)
            pltpu.async_copy(rels_hbm.at[rmv[j], rsv[j]], r_v.at[e], sem)
            pltpu.async_copy(tails_hbm.at[tmv[j], tsv[j]], t_v.at[e], sem)
            pltpu.async_copy(names_hbm.at[nmv[j], nsv[j]], n_v.at[e], sem)
        return carry

    lax.fori_loop(0, _NG, issue_body, 0)

    def drain_body(e, carry):
        pltpu.make_async_copy(heads_hbm.at[0, 0], h_v.at[0], sem).wait()
        pltpu.make_async_copy(rels_hbm.at[0, 0], r_v.at[0], sem).wait()
        pltpu.make_async_copy(tails_hbm.at[0, 0], t_v.at[0], sem).wait()
        pltpu.make_async_copy(names_hbm.at[0, 0], n_v.at[0], sem).wait()
        return carry

    lax.fori_loop(0, _B, drain_body, 0)

    @pl.when(wid == _NW - 1)
    def _point_question_tail_at_mask():
        pltpu.sync_copy(specials_hbm.at[0, 0], t_v.at[_B - 1])

    def grp_body(g, carry):
        e0 = g * 16
        ho16 = (hid_v[pl.ds(e0, 16)] & 1) << 6
        ro16 = (rid_v[pl.ds(e0, 16)] & 1) << 6
        to16 = (tid_v[pl.ds(e0, 16)] & 1) << 6
        no16 = (nid_v[pl.ds(e0, 16)] & 1) << 6
        for j in range(16):
            r = e0 + j
            ho, ro, to, no = ho16[j], ro16[j], to16[j], no16[j]
            for c in range(_EMB // 16):
                s = 16 * c
                n = n_v[r, pl.ds(no + s, 16)]
                out_v[r, pl.ds(s, 16)] = n + h_v[r, pl.ds(ho + s, 16)]
                out_v[r, pl.ds(_EMB + s, 16)] = r_v[r, pl.ds(ro + s, 16)]
                out_v[r, pl.ds(2 * _EMB + s, 16)] = (
                    n + t_v[r, pl.ds(to + s, 16)])
        return carry

    lax.fori_loop(0, _NG, grp_body, 0)

    pltpu.sync_copy(out_v, out_hbm.at[pl.ds(base, _B)])


def kernel(heads_w, relations_w, tails_w, names_w, specials_w,
           head_idx, rel_idx, tail_idx, name_idx, q_head, q_rel, q_name):
    i32 = jnp.int32
    hid = jnp.concatenate([head_idx.astype(i32), q_head.astype(i32)])
    rid = jnp.concatenate([rel_idx.astype(i32), q_rel.astype(i32)])
    tid = jnp.concatenate([tail_idx.astype(i32), jnp.ones((1,), i32)])
    nid = jnp.concatenate([name_idx.astype(i32), q_name.astype(i32)])
    heads3 = heads_w.reshape(-1, 8, _LINE)
    rels3 = jnp.pad(relations_w, ((0, 8), (0, 0))).reshape(-1, 8, _LINE)
    tails3 = tails_w.reshape(-1, 8, _LINE)
    names3 = names_w.reshape(-1, 8, _LINE)
    specials3 = jnp.pad(specials_w, ((0, 0), (0, 0))).reshape(1, 1, _LINE)
    specials3 = jnp.pad(specials3, ((0, 0), (0, 7), (0, 0)))
    return _emb_kernel(heads3, rels3, tails3, names3, specials3,
                       hid, rid, tid, nid)


# speedup vs baseline: 1.7423x; 1.7423x over previous
import functools

import jax
import jax.numpy as jnp
from jax import lax
from jax.experimental import pallas as pl
from jax.experimental.pallas import tpu as pltpu
from jax.experimental.pallas import tpu_sc as plsc

_NUM_ROWS = 4096
_EMB = 64
_NUM_COLS = 3 * _EMB
_NC = 2
_NS = 16
_NW = _NC * _NS
_B = _NUM_ROWS // _NW
_NG = _B // 16


@functools.partial(
    pl.kernel,
    mesh=plsc.VectorSubcoreMesh(core_axis_name="c", subcore_axis_name="s"),
    out_type=jax.ShapeDtypeStruct((_NUM_ROWS, _NUM_COLS), jnp.float32),
    scratch_types=[
        pltpu.VMEM((_B,), jnp.int32),
        pltpu.VMEM((_B,), jnp.int32),
        pltpu.VMEM((_B,), jnp.int32),
        pltpu.VMEM((_B,), jnp.int32),
        pltpu.VMEM((_B, _EMB), jnp.float32),
        pltpu.VMEM((_B, _EMB), jnp.float32),
        pltpu.VMEM((_B, _EMB), jnp.float32),
        pltpu.VMEM((_B, _EMB), jnp.float32),
        pltpu.VMEM((_B, _NUM_COLS), jnp.float32),
        pltpu.SemaphoreType.DMA,
    ],
)
def _emb_kernel(heads_hbm, rels_hbm, tails_hbm, names_hbm, specials_hbm,
                hid_hbm, rid_hbm, tid_hbm, nid_hbm, out_hbm,
                hid_v, rid_v, tid_v, nid_v,
                h_v, r_v, t_v, n_v, out_v, sem):
    wid = lax.axis_index("s") * _NC + lax.axis_index("c")
    base = wid * _B

    pltpu.sync_copy(hid_hbm.at[pl.ds(base, _B)], hid_v)
    pltpu.sync_copy(rid_hbm.at[pl.ds(base, _B)], rid_v)
    pltpu.sync_copy(tid_hbm.at[pl.ds(base, _B)], tid_v)
    pltpu.sync_copy(nid_hbm.at[pl.ds(base, _B)], nid_v)

    def issue_body(g, carry):
        e0 = g * 16
        hv = hid_v[pl.ds(e0, 16)]
        rv = rid_v[pl.ds(e0, 16)]
        tv = tid_v[pl.ds(e0, 16)]
        nv = nid_v[pl.ds(e0, 16)]
        htv, hsv = hv >> 3, hv & 7
        rtv, rsv = rv >> 3, rv & 7
        ttv, tsv = tv >> 3, tv & 7
        ntv, nsv = nv >> 3, nv & 7
        for j in range(16):
            e = e0 + j
            pltpu.async_copy(heads_hbm.at[htv[j], hsv[j]], h_v.at[e], sem)
            pltpu.async_copy(rels_hbm.at[rtv[j], rsv[j]], r_v.at[e], sem)
            pltpu.async_copy(tails_hbm.at[ttv[j], tsv[j]], t_v.at[e], sem)
            pltpu.async_copy(names_hbm.at[ntv[j], nsv[j]], n_v.at[e], sem)
        return carry

    lax.fori_loop(0, _NG, issue_body, 0)

    def drain_body(e, carry):
        pltpu.make_async_copy(heads_hbm.at[0, 0], h_v.at[0], sem).wait()
        pltpu.make_async_copy(rels_hbm.at[0, 0], r_v.at[0], sem).wait()
        pltpu.make_async_copy(tails_hbm.at[0, 0], t_v.at[0], sem).wait()
        pltpu.make_async_copy(names_hbm.at[0, 0], n_v.at[0], sem).wait()
        return carry

    lax.fori_loop(0, _B, drain_body, 0)

    @pl.when(wid == _NW - 1)
    def _point_question_tail_at_mask():
        pltpu.sync_copy(specials_hbm.at[0, 1], t_v.at[_B - 1])

    def row_body(r, carry):
        for c in range(_EMB // 16):
            s = 16 * c
            n = n_v[r, pl.ds(s, 16)]
            out_v[r, pl.ds(s, 16)] = n + h_v[r, pl.ds(s, 16)]
            out_v[r, pl.ds(_EMB + s, 16)] = r_v[r, pl.ds(s, 16)]
            out_v[r, pl.ds(2 * _EMB + s, 16)] = n + t_v[r, pl.ds(s, 16)]
        return carry

    lax.fori_loop(0, _B, row_body, 0)

    pltpu.sync_copy(out_v, out_hbm.at[pl.ds(base, _B)])


def kernel(heads_w, relations_w, tails_w, names_w, specials_w,
           head_idx, rel_idx, tail_idx, name_idx, q_head, q_rel, q_name):
    i32 = jnp.int32
    hid = jnp.concatenate([head_idx.astype(i32), q_head.astype(i32)])
    rid = jnp.concatenate([rel_idx.astype(i32), q_rel.astype(i32)])
    tid = jnp.concatenate([tail_idx.astype(i32), jnp.zeros((1,), i32)])
    nid = jnp.concatenate([name_idx.astype(i32), q_name.astype(i32)])
    heads3 = heads_w.reshape(-1, 8, _EMB)
    rels3 = relations_w.reshape(-1, 8, _EMB)
    tails3 = tails_w.reshape(-1, 8, _EMB)
    names3 = names_w.reshape(-1, 8, _EMB)
    specials3 = jnp.pad(specials_w, ((0, 6), (0, 0))).reshape(1, 8, _EMB)
    return _emb_kernel(heads3, rels3, tails3, names3, specials3,
                       hid, rid, tid, nid)
